# baseline (speedup 1.0000x reference)
import functools

import jax
import jax.numpy as jnp
from jax import lax
from jax.experimental import pallas as pl
from jax.experimental.pallas import tpu as pltpu
from jax.experimental.pallas import tpu_sc as plsc

D_MODEL = 768
NUM_ROWS = 6
BATCH = 16384

_info = plsc.get_sparse_core_info()
NUM_CORES = _info.num_cores
NUM_SUBCORES = _info.num_subcores
NUM_WORKERS = NUM_CORES * NUM_SUBCORES
B_PER_W = BATCH // NUM_WORKERS
CHUNK = 128
N_CHUNKS = B_PER_W // CHUNK

_mesh = plsc.VectorSubcoreMesh(core_axis_name="c", subcore_axis_name="s")


@functools.partial(
    pl.kernel,
    mesh=_mesh,
    out_type=jax.ShapeDtypeStruct((BATCH, D_MODEL), jnp.float32),
    scratch_types=[
        pltpu.VMEM((N_CHUNKS, CHUNK), jnp.int32),
        pltpu.VMEM((CHUNK, D_MODEL), jnp.float32),
        pltpu.SemaphoreType.DMA,
    ],
)
def _emb_kernel(idx_hbm, table_hbm, out_hbm, idx_v, rows_v, sem):
    cid = lax.axis_index("c")
    sid = lax.axis_index("s")
    wid = sid * NUM_CORES + cid
    base = wid * B_PER_W

    pltpu.sync_copy(idx_hbm.at[wid], idx_v)

    for ci in range(N_CHUNKS):
        pltpu.async_copy(table_hbm.at[idx_v.at[ci]], rows_v, sem).wait()
        pltpu.sync_copy(rows_v, out_hbm.at[pl.ds(base + ci * CHUNK, CHUNK)])


def kernel(emotion_id, table):
    if emotion_id.ndim > 1:
        emotion_id = emotion_id.reshape(-1)
    idx = emotion_id.astype(jnp.int32).reshape(NUM_WORKERS, N_CHUNKS, CHUNK)
    return _emb_kernel(idx, table)


# speedup vs baseline: 3.5449x; 3.5449x over previous
import functools

import jax
import jax.numpy as jnp
from jax import lax
from jax.experimental import pallas as pl
from jax.experimental.pallas import tpu as pltpu
from jax.experimental.pallas import tpu_sc as plsc

D_MODEL = 768
NUM_ROWS = 6
BATCH = 16384

_info = plsc.get_sparse_core_info()
NUM_CORES = _info.num_cores
NUM_SUBCORES = _info.num_subcores
NUM_WORKERS = NUM_CORES * NUM_SUBCORES
B_PER_W = BATCH // NUM_WORKERS
CHUNK = 64
N_CHUNKS = B_PER_W // CHUNK
NBUF = 2

_mesh = plsc.VectorSubcoreMesh(core_axis_name="c", subcore_axis_name="s")


@functools.partial(
    pl.kernel,
    mesh=_mesh,
    out_type=jax.ShapeDtypeStruct((BATCH, D_MODEL), jnp.float32),
    scratch_types=[
        pltpu.VMEM((N_CHUNKS, CHUNK), jnp.int32),
        pltpu.VMEM((NBUF, CHUNK, D_MODEL), jnp.float32),
        pltpu.SemaphoreType.DMA,
        pltpu.SemaphoreType.DMA,
        pltpu.SemaphoreType.DMA,
    ],
)
def _emb_kernel(idx_hbm, table_hbm, out_hbm, idx_v, rows_v, g0, g1, wsem):
    cid = lax.axis_index("c")
    sid = lax.axis_index("s")
    wid = sid * NUM_CORES + cid
    base = wid * B_PER_W
    gsems = (g0, g1)

    pltpu.sync_copy(idx_hbm.at[wid], idx_v)

    pltpu.make_async_copy(
        table_hbm.at[idx_v.at[0]], rows_v.at[0], gsems[0]
    ).start()

    for ci in range(N_CHUNKS):
        slot = ci % NBUF
        if ci >= 1:
            pltpu.make_async_copy(
                rows_v.at[1 - slot],
                out_hbm.at[pl.ds(base + (ci - 1) * CHUNK, CHUNK)],
                wsem,
            ).wait()
        if ci + 1 < N_CHUNKS:
            pltpu.make_async_copy(
                table_hbm.at[idx_v.at[ci + 1]], rows_v.at[1 - slot],
                gsems[1 - slot],
            ).start()
        pltpu.make_async_copy(
            table_hbm.at[idx_v.at[ci]], rows_v.at[slot], gsems[slot]
        ).wait()
        pltpu.make_async_copy(
            rows_v.at[slot],
            out_hbm.at[pl.ds(base + ci * CHUNK, CHUNK)],
            wsem,
        ).start()

    pltpu.make_async_copy(
        rows_v.at[(N_CHUNKS - 1) % NBUF],
        out_hbm.at[pl.ds(base + (N_CHUNKS - 1) * CHUNK, CHUNK)],
        wsem,
    ).wait()


def kernel(emotion_id, table):
    if emotion_id.ndim > 1:
        emotion_id = emotion_id.reshape(-1)
    idx = emotion_id.astype(jnp.int32).reshape(NUM_WORKERS, N_CHUNKS, CHUNK)
    idx = idx + (jnp.arange(NUM_WORKERS, dtype=jnp.int32) * NUM_ROWS)[:, None, None]
    table_rep = jnp.tile(table, (NUM_WORKERS, 1))
    return _emb_kernel(idx, table_rep)


# speedup vs baseline: 6.7045x; 1.8913x over previous
import functools

import jax
import jax.numpy as jnp
from jax import lax
from jax.experimental import pallas as pl
from jax.experimental.pallas import tpu as pltpu
from jax.experimental.pallas import tpu_sc as plsc

D_MODEL = 768
NUM_ROWS = 6
BATCH = 16384

_info = plsc.get_sparse_core_info()
NUM_CORES = _info.num_cores
NUM_SUBCORES = _info.num_subcores
NUM_WORKERS = NUM_CORES * NUM_SUBCORES
B_PER_W = BATCH // NUM_WORKERS
LANES = 16
N_GROUPS = B_PER_W // LANES

_mesh = plsc.VectorSubcoreMesh(core_axis_name="c", subcore_axis_name="s")


@functools.partial(
    pl.kernel,
    mesh=_mesh,
    out_type=jax.ShapeDtypeStruct((BATCH, D_MODEL), jnp.float32),
    scratch_types=[
        pltpu.VMEM((B_PER_W,), jnp.int32),
        pltpu.VMEM((NUM_ROWS, D_MODEL), jnp.float32),
        pltpu.SemaphoreType.DMA,
    ],
)
def _emb_kernel(idx_hbm, table_hbm, out_hbm, idx_v, table_v, wsem):
    cid = lax.axis_index("c")
    sid = lax.axis_index("s")
    wid = sid * NUM_CORES + cid
    base = wid * B_PER_W

    pltpu.sync_copy(table_hbm, table_v)
    pltpu.sync_copy(idx_hbm.at[wid], idx_v)

    def group_body(g, _):
        v = idx_v[pl.ds(g * LANES, LANES)]
        b = base + g * LANES
        for l in range(LANES):
            e = v[l]
            pltpu.make_async_copy(
                table_v.at[e], out_hbm.at[b + l], wsem
            ).start()
        return 0

    lax.fori_loop(0, N_GROUPS, group_body, 0)

    pltpu.make_async_copy(
        out_hbm.at[pl.ds(base, B_PER_W)],
        out_hbm.at[pl.ds(base, B_PER_W)],
        wsem,
    ).wait()


def kernel(emotion_id, table):
    if emotion_id.ndim > 1:
        emotion_id = emotion_id.reshape(-1)
    idx = emotion_id.astype(jnp.int32).reshape(NUM_WORKERS, B_PER_W)
    return _emb_kernel(idx, table)


# speedup vs baseline: 9.2310x; 1.3768x over previous
import functools

import jax
import jax.numpy as jnp
from jax.experimental import pallas as pl
from jax.experimental.pallas import tpu as pltpu

D_MODEL = 768
NUM_ROWS = 6
BATCH = 16384

BLK = 512
NBLK = BATCH // BLK


def _tc_body(idx_ref, table_ref, out_ref):
    idx = idx_ref[0, 0, :]
    onehot = (idx[:, None] == jax.lax.broadcasted_iota(jnp.int32, (1, NUM_ROWS), 1)).astype(jnp.float32)
    out_ref[...] = jnp.dot(onehot, table_ref[...], preferred_element_type=jnp.float32)


@jax.jit
def _tc_lookup(idx3, table):
    return pl.pallas_call(
        _tc_body,
        grid=(NBLK,),
        in_specs=[
            pl.BlockSpec((1, 1, BLK), lambda i: (i, 0, 0)),
            pl.BlockSpec((NUM_ROWS, D_MODEL), lambda i: (0, 0)),
        ],
        out_specs=pl.BlockSpec((BLK, D_MODEL), lambda i: (i, 0)),
        out_shape=jax.ShapeDtypeStruct((BATCH, D_MODEL), jnp.float32),
    )(idx3, table)


def kernel(emotion_id, table):
    if emotion_id.ndim > 1:
        emotion_id = emotion_id.reshape(-1)
    idx3 = emotion_id.astype(jnp.int32).reshape(NBLK, 1, BLK)
    return _tc_lookup(idx3, table)
